# baseline (speedup 1.0000x reference)
import functools

import jax
import jax.numpy as jnp
from jax.experimental import pallas as pl
from jax.experimental.pallas import tpu as pltpu

_L = 256
_MAX_M = 512


def _autoenc_body(x_ref, w0_ref, wp_ref, w2c_ref, wu4_ref, wu6_ref, bp_ref,
                  wenc_ref, benc_ref, wdec_ref, bdec_ref, out_ref):
    f32 = jnp.float32
    bf16 = jnp.bfloat16

    zcache = {}

    def zeros(shape):
        if shape not in zcache:
            zcache[shape] = jnp.zeros(shape, bf16)
        return zcache[shape]

    def conv3x3(srcs, wgetters, bias, relu, out_dtype=bf16, fuse_pool=False):
        Bb, H, W, C = srcs[0].shape
        Hs = H
        while Hs > 1 and Hs % 2 == 0 and Bb * Hs * W > _MAX_M:
            Hs //= 2
        M = Bb * Hs * W
        zcol = zeros((Bb, H, 1, C))

        shifted = []
        for s in srcs:
            sm = jnp.concatenate([zcol, s[:, :, :W - 1, :]], axis=2)
            sp = jnp.concatenate([s[:, :, 1:, :], zcol], axis=2)
            shifted.append((sm, s, sp))

        def rows(ab, lo):
            hi = lo + Hs
            parts = []
            if lo < 0:
                parts.append(zeros((Bb, -lo, W, C)))
            lo_c, hi_c = max(lo, 0), min(hi, H)
            if hi_c > lo_c:
                parts.append(ab[:, lo_c:hi_c])
            if hi > H:
                parts.append(zeros((Bb, hi - H, W, C)))
            return parts[0] if len(parts) == 1 else jnp.concatenate(parts, axis=1)

        nout = len(bias)
        outs = [[] for _ in bias]
        for h0 in range(0, H, Hs):
            acc = None
            for dy in range(3):
                for si in range(len(srcs)):
                    for dx in range(3):
                        tap = rows(shifted[si][dx], h0 + dy - 1)
                        t2 = tap.reshape(M, C)
                        part = jnp.dot(t2, wgetters[si](dy * 3 + dx),
                                       preferred_element_type=f32)
                        acc = part if acc is None else acc + part
            for j in range(nout):
                y = acc[:, j * _L:(j + 1) * _L] + bias[j]
                if relu:
                    y = jnp.maximum(y, 0.0)
                if fuse_pool:
                    y5 = y.reshape(Bb, Hs // 2, 2, W, _L)
                    v = jnp.maximum(y5[:, :, 0], y5[:, :, 1])
                    s = v.reshape(Bb, Hs // 2, W // 2, 2, _L)
                    y4 = jnp.maximum(s[:, :, :, 0, :], s[:, :, :, 1, :])
                    outs[j].append(y4.astype(out_dtype))
                else:
                    outs[j].append(y.astype(out_dtype).reshape(Bb, Hs, W, _L))
        return [o[0] if len(o) == 1 else jnp.concatenate(o, axis=1) for o in outs]

    def conv_up(srcs, wgetters, bias, relu, phase_out=False):
        Bb, H, W, C = srcs[0].shape
        Hs = H
        while Hs > 1 and Hs % 2 == 0 and Bb * Hs * W > _MAX_M:
            Hs //= 2
        M = Bb * Hs * W
        zcol = zeros((Bb, H, 1, C))
        shifted = []
        for s in srcs:
            sm = jnp.concatenate([zcol, s[:, :, :W - 1, :]], axis=2)
            sp = jnp.concatenate([s[:, :, 1:, :], zcol], axis=2)
            shifted.append((sm, s, sp))

        def rows(ab, lo):
            hi = lo + Hs
            parts = []
            if lo < 0:
                parts.append(zeros((Bb, -lo, W, C)))
            lo_c, hi_c = max(lo, 0), min(hi, H)
            if hi_c > lo_c:
                parts.append(ab[:, lo_c:hi_c])
            if hi > H:
                parts.append(zeros((Bb, hi - H, W, C)))
            return parts[0] if len(parts) == 1 else jnp.concatenate(parts, axis=1)

        phases = []
        for py in (0, 1):
            for px in (0, 1):
                strips = []
                for h0 in range(0, H, Hs):
                    acc = None
                    for syi, sy in enumerate((-1, 0) if py == 0 else (0, 1)):
                        for si in range(len(srcs)):
                            for sxi, sx in enumerate((-1, 0) if px == 0 else (0, 1)):
                                tap = rows(shifted[si][sx + 1], h0 + sy)
                                t2 = tap.reshape(M, C)
                                part = jnp.dot(
                                    t2, wgetters[si](py * 2 + px, syi * 2 + sxi),
                                    preferred_element_type=f32)
                                acc = part if acc is None else acc + part
                    y = acc + bias
                    if relu:
                        y = jnp.maximum(y, 0.0)
                    strips.append(y.astype(bf16).reshape(Bb, Hs, W, _L))
                phases.append(strips[0] if len(strips) == 1
                              else jnp.concatenate(strips, axis=1))
        if phase_out:
            return phases
        p00, p01, p10, p11 = phases

        def ilw(u, v):
            return jnp.stack([u, v], axis=3).reshape(Bb, H, 2 * W, _L)

        r0, r1 = ilw(p00, p01), ilw(p10, p11)
        return jnp.stack([r0, r1], axis=2).reshape(Bb, 2 * H, 2 * W, _L)

    def wp(i):
        return lambda t, i=i: wp_ref[i, t]

    def conv0(xv):
        Bb, H, W, C = xv.shape
        Hs = H
        while Hs > 1 and Hs % 2 == 0 and Bb * Hs * W > _MAX_M:
            Hs //= 2
        M = Bb * Hs * W
        zcol = zeros((Bb, H, 1, C))
        zrow = zeros((Bb, 1, W, C))
        sm = jnp.concatenate([zcol, xv[:, :, :W - 1, :]], axis=2)
        sp = jnp.concatenate([xv[:, :, 1:, :], zcol], axis=2)
        taps9 = []
        for dy in range(3):
            for s in (sm, xv, sp):
                if dy == 0:
                    t = jnp.concatenate([zrow, s[:, :H - 1]], axis=1)
                elif dy == 1:
                    t = s
                else:
                    t = jnp.concatenate([s[:, 1:], zrow], axis=1)
                taps9.append(t)
        taps9.append(zeros((Bb, H, W, _L - 9 * C)))
        cat = jnp.concatenate(taps9, axis=3)
        w0c = w0_ref[...]
        strips = []
        for h0 in range(0, H, Hs):
            t2 = cat[:, h0:h0 + Hs].reshape(M, _L)
            y = jnp.dot(t2, w0c, preferred_element_type=f32) + bp_ref[0]
            y = jnp.maximum(y, 0.0)
            strips.append(y.astype(bf16).reshape(Bb, Hs, W, _L))
        return strips[0] if len(strips) == 1 else jnp.concatenate(strips, axis=1)

    bt4 = x_ref.shape[0]
    bt2 = 2 * bt4

    xv = x_ref[...]
    a = conv0(xv)
    a, = conv3x3([a], [wp(0)], [bp_ref[1]], relu=True,
                 fuse_pool=True)

    oa, ob = conv3x3([a], [lambda t: w2c_ref[t]],
                     [bp_ref[2], bp_ref[2]], relu=True)
    a2 = jnp.stack([oa, ob], axis=1).reshape(bt2, 8, 8, _L)
    a2, = conv3x3([a2], [wp(1)], [bp_ref[3]], relu=True,
                  fuse_pool=True)

    flat = a2.reshape(bt2, 16 * _L)
    z = jnp.dot(flat, wenc_ref[...], preferred_element_type=f32) + benc_ref[...]
    o = jnp.dot(z.astype(bf16), wdec_ref[...], preferred_element_type=f32)
    o = jnp.maximum(o + bdec_ref[...], 0.0).astype(bf16)
    a2 = o.reshape(bt2, 4, 4, _L)

    r = a2.reshape(bt4, 2, 4, 4, _L)
    yev, yod = r[:, 0], r[:, 1]
    a = conv_up([yev, yod],
                [lambda p, k: wu4_ref[0, p, k], lambda p, k: wu4_ref[1, p, k]],
                bp_ref[4], relu=True)
    a, = conv3x3([a], [wp(2)], [bp_ref[5]], relu=True)
    a = conv_up([a], [lambda p, k: wu6_ref[p, k]],
                bp_ref[6], relu=True)
    a, = conv3x3([a], [wp(3)], [bp_ref[7]], relu=True)
    a, = conv3x3([a], [wp(4)], [bp_ref[8]], relu=False,
                 out_dtype=f32)

    out_ref[...] = jnp.concatenate(
        [a[..., 0:4], a[..., 64:68], a[..., 128:132], a[..., 192:196]], axis=-1)


def _fold_up(taps):
    dmap = {(0, -1): (0,), (0, 0): (1, 2), (1, 0): (0, 1), (1, 1): (2,)}
    phases = []
    for py in (0, 1):
        for px in (0, 1):
            ktaps = []
            for sy in ((-1, 0) if py == 0 else (0, 1)):
                for sx in ((-1, 0) if px == 0 else (0, 1)):
                    w = 0
                    for dy in dmap[(py, sy)]:
                        for dx in dmap[(px, sx)]:
                            w = w + taps[dy * 3 + dx]
                    ktaps.append(w)
            phases.append(jnp.stack(ktaps))
    return jnp.stack(phases).astype(jnp.bfloat16)


def _pack_params(wconv, bconv, wenc, benc, wdec, bdec):
    bf16 = jnp.bfloat16
    f32 = jnp.float32
    z9 = jnp.zeros((9, _L, _L), bf16)

    def diag4(i):
        w = wconv[i][:, :64, :64]
        out = z9
        for e in range(4):
            out = out.at[:, 64 * e:64 * e + 64, 64 * e:64 * e + 64].set(w)
        return out

    def diag2(i):
        w = wconv[i][:, :128, :128]
        return z9.at[:, 0:128, 0:128].set(w).at[:, 128:256, 128:256].set(w)

    w2 = wconv[2][:, :64, :128]
    w2a = z9.at[:, 0:64, 0:128].set(w2).at[:, 64:128, 128:256].set(w2)
    w2b = z9.at[:, 128:192, 0:128].set(w2).at[:, 192:256, 128:256].set(w2)
    w2c = jnp.concatenate([w2a, w2b], axis=2)
    wp = jnp.stack([diag4(1), diag2(3), diag4(5), diag4(7), diag4(8)])

    z9f = jnp.zeros((9, _L, _L), f32)
    w4 = wconv[4][:, :128, :64].astype(f32)
    w4a = z9f.at[:, 0:128, 0:64].set(w4).at[:, 128:256, 64:128].set(w4)
    w4b = z9f.at[:, 0:128, 128:192].set(w4).at[:, 128:256, 192:256].set(w4)
    wu4 = jnp.stack([_fold_up(w4a), _fold_up(w4b)])
    w6 = wconv[6][:, :64, :64].astype(f32)
    w6d = jnp.zeros((9, _L, _L), f32)
    for e in range(4):
        w6d = w6d.at[:, 64 * e:64 * e + 64, 64 * e:64 * e + 64].set(w6)
    wu6 = _fold_up(w6d)

    w0s = wconv[0][:, :4, :64]
    w0 = jnp.zeros((9, 16, _L), bf16)
    for e in range(4):
        w0 = w0.at[:, 4 * e:4 * e + 4, 64 * e:64 * e + 64].set(w0s)
    w0 = jnp.pad(w0.reshape(9 * 16, _L), ((0, _L - 9 * 16), (0, 0)))

    def tile4(v):
        return jnp.concatenate([v[:64]] * 4)

    def tile2(v):
        return jnp.concatenate([v[:128]] * 2)

    bp = jnp.stack([tile4(bconv[0]), tile4(bconv[1]), tile2(bconv[2]),
                    tile2(bconv[3]), tile4(bconv[4]), tile4(bconv[5]),
                    tile4(bconv[6]), tile4(bconv[7]), tile4(bconv[8])]).astype(f32)

    z16 = jnp.zeros((16, _L, _L), bf16)
    wenc2 = z16.at[:, :128, :128].set(wenc).at[:, 128:, 128:].set(wenc)
    wenc2 = wenc2.reshape(16 * _L, _L)
    benc2 = jnp.concatenate([benc, benc], axis=1).astype(f32)
    wdec2 = z16.at[:, :128, :128].set(wdec).at[:, 128:, 128:].set(wdec)
    wdec2 = wdec2.transpose(1, 0, 2).reshape(_L, 16 * _L)
    bdec2 = jnp.concatenate([bdec, bdec], axis=1).reshape(1, 16 * _L).astype(f32)
    return w0, wp, w2c, wu4, wu6, bp, wenc2, benc2, wdec2, bdec2


def kernel(wconv, bconv, wenc, benc, wdec, bdec, x_nchw):
    B, C, H, W = x_nchw.shape
    for bt in (32, 16, 8, 4):
        if B % bt == 0:
            Bt = bt
            break
    else:
        raise ValueError("batch must be divisible by 4")
    bt4 = Bt // 4

    w0, wp, w2c, wu4, wu6, bp, wenc2, benc2, wdec2, bdec2 = _pack_params(
        wconv, bconv, wenc, benc, wdec, bdec)

    xt = jnp.transpose(x_nchw, (0, 2, 3, 1)).astype(jnp.bfloat16)
    xt = jnp.pad(xt, ((0, 0), (0, 0), (0, 0), (0, 4 - C)))
    x4 = xt.reshape(B // 4, 4, H, W, 4).transpose(0, 2, 3, 1, 4)
    x4 = x4.reshape(B // 4, H, W, 16)

    out = pl.pallas_call(
        _autoenc_body,
        out_shape=jax.ShapeDtypeStruct((B // 4, H, W, 16), jnp.float32),
        grid=(B // Bt,),
        in_specs=[
            pl.BlockSpec((bt4, H, W, 16), lambda b: (b, 0, 0, 0)),
            pl.BlockSpec((_L, _L), lambda b: (0, 0)),
            pl.BlockSpec((5, 9, _L, _L), lambda b: (0, 0, 0, 0)),
            pl.BlockSpec((9, _L, 2 * _L), lambda b: (0, 0, 0)),
            pl.BlockSpec((2, 4, 4, _L, _L), lambda b: (0, 0, 0, 0, 0)),
            pl.BlockSpec((4, 4, _L, _L), lambda b: (0, 0, 0, 0)),
            pl.BlockSpec((9, _L), lambda b: (0, 0)),
            pl.BlockSpec((16 * _L, _L), lambda b: (0, 0)),
            pl.BlockSpec((1, _L), lambda b: (0, 0)),
            pl.BlockSpec((_L, 16 * _L), lambda b: (0, 0)),
            pl.BlockSpec((1, 16 * _L), lambda b: (0, 0)),
        ],
        out_specs=pl.BlockSpec((bt4, H, W, 16), lambda b: (b, 0, 0, 0)),
        compiler_params=pltpu.CompilerParams(
            dimension_semantics=("parallel",),
            vmem_limit_bytes=48 * 1024 * 1024,
        ),
    )(x4, w0, wp, w2c, wu4, wu6, bp, wenc2, benc2, wdec2, bdec2)

    y = out.reshape(B // 4, H, W, 4, 4)[..., :C]
    return y.transpose(0, 3, 4, 1, 2).reshape(B, C, H, W)


# speedup vs baseline: 1.0876x; 1.0876x over previous
import jax
import jax.numpy as jnp
from jax.experimental import pallas as pl
from jax.experimental.pallas import tpu as pltpu

_L = 256
_MAX_M = 256


def _autoenc_body(x_ref, w0_ref, wp_ref, w2c_ref, wu4_ref, wu6_ref, bp_ref,
                  wenc_ref, benc_ref, wdec_ref, bdec_ref, out_ref):
    f32 = jnp.float32
    bf16 = jnp.bfloat16

    zcache = {}

    def zeros(shape):
        if shape not in zcache:
            zcache[shape] = jnp.zeros(shape, bf16)
        return zcache[shape]

    def conv3x3(srcs, wgetters, bias, relu, out_dtype=bf16, fuse_pool=False):
        Bb, H, W, C = srcs[0].shape
        Hs = H
        while Hs > 1 and Hs % 2 == 0 and Bb * Hs * W > _MAX_M:
            Hs //= 2
        M = Bb * Hs * W
        zcol = zeros((Bb, H, 1, C))

        shifted = []
        for s in srcs:
            sm = jnp.concatenate([zcol, s[:, :, :W - 1, :]], axis=2)
            sp = jnp.concatenate([s[:, :, 1:, :], zcol], axis=2)
            shifted.append((sm, s, sp))

        def rows(ab, lo):
            hi = lo + Hs
            parts = []
            if lo < 0:
                parts.append(zeros((Bb, -lo, W, C)))
            lo_c, hi_c = max(lo, 0), min(hi, H)
            if hi_c > lo_c:
                parts.append(ab[:, lo_c:hi_c])
            if hi > H:
                parts.append(zeros((Bb, hi - H, W, C)))
            return parts[0] if len(parts) == 1 else jnp.concatenate(parts, axis=1)

        nout = len(bias)
        outs = [[] for _ in bias]
        for h0 in range(0, H, Hs):
            acc = None
            for dy in range(3):
                for si in range(len(srcs)):
                    for dx in range(3):
                        tap = rows(shifted[si][dx], h0 + dy - 1)
                        t2 = tap.reshape(M, C)
                        part = jnp.dot(t2, wgetters[si](dy * 3 + dx),
                                       preferred_element_type=f32)
                        acc = part if acc is None else acc + part
            for j in range(nout):
                y = acc[:, j * _L:(j + 1) * _L] + bias[j]
                if relu:
                    y = jnp.maximum(y, 0.0)
                if fuse_pool:
                    y5 = y.reshape(Bb, Hs // 2, 2, W, _L)
                    v = jnp.maximum(y5[:, :, 0], y5[:, :, 1])
                    s = v.reshape(Bb, Hs // 2, W // 2, 2, _L)
                    y4 = jnp.maximum(s[:, :, :, 0, :], s[:, :, :, 1, :])
                    outs[j].append(y4.astype(out_dtype))
                else:
                    outs[j].append(y.astype(out_dtype).reshape(Bb, Hs, W, _L))
        return [o[0] if len(o) == 1 else jnp.concatenate(o, axis=1) for o in outs]

    def conv_up(srcs, wgetters, bias, relu, phase_out=False):
        Bb, H, W, C = srcs[0].shape
        Hs = H
        while Hs > 1 and Hs % 2 == 0 and Bb * Hs * W > _MAX_M:
            Hs //= 2
        M = Bb * Hs * W
        zcol = zeros((Bb, H, 1, C))
        shifted = []
        for s in srcs:
            sm = jnp.concatenate([zcol, s[:, :, :W - 1, :]], axis=2)
            sp = jnp.concatenate([s[:, :, 1:, :], zcol], axis=2)
            shifted.append((sm, s, sp))

        def rows(ab, lo):
            hi = lo + Hs
            parts = []
            if lo < 0:
                parts.append(zeros((Bb, -lo, W, C)))
            lo_c, hi_c = max(lo, 0), min(hi, H)
            if hi_c > lo_c:
                parts.append(ab[:, lo_c:hi_c])
            if hi > H:
                parts.append(zeros((Bb, hi - H, W, C)))
            return parts[0] if len(parts) == 1 else jnp.concatenate(parts, axis=1)

        phases = []
        for py in (0, 1):
            for px in (0, 1):
                strips = []
                for h0 in range(0, H, Hs):
                    acc = None
                    for syi, sy in enumerate((-1, 0) if py == 0 else (0, 1)):
                        for si in range(len(srcs)):
                            for sxi, sx in enumerate((-1, 0) if px == 0 else (0, 1)):
                                tap = rows(shifted[si][sx + 1], h0 + sy)
                                t2 = tap.reshape(M, C)
                                part = jnp.dot(
                                    t2, wgetters[si](py * 2 + px, syi * 2 + sxi),
                                    preferred_element_type=f32)
                                acc = part if acc is None else acc + part
                    y = acc + bias
                    if relu:
                        y = jnp.maximum(y, 0.0)
                    strips.append(y.astype(bf16).reshape(Bb, Hs, W, _L))
                phases.append(strips[0] if len(strips) == 1
                              else jnp.concatenate(strips, axis=1))
        if phase_out:
            return phases
        p00, p01, p10, p11 = phases

        def ilw(u, v):
            return jnp.stack([u, v], axis=3).reshape(Bb, H, 2 * W, _L)

        r0, r1 = ilw(p00, p01), ilw(p10, p11)
        return jnp.stack([r0, r1], axis=2).reshape(Bb, 2 * H, 2 * W, _L)

    def wp(i):
        return lambda t, i=i: wp_ref[i, t]

    def conv0(xv):
        Bb, H, W, C = xv.shape
        Hs = H
        while Hs > 1 and Hs % 2 == 0 and Bb * Hs * W > _MAX_M:
            Hs //= 2
        M = Bb * Hs * W
        zcol = zeros((Bb, H, 1, C))
        zrow = zeros((Bb, 1, W, C))
        sm = jnp.concatenate([zcol, xv[:, :, :W - 1, :]], axis=2)
        sp = jnp.concatenate([xv[:, :, 1:, :], zcol], axis=2)
        taps9 = []
        for dy in range(3):
            for s in (sm, xv, sp):
                if dy == 0:
                    t = jnp.concatenate([zrow, s[:, :H - 1]], axis=1)
                elif dy == 1:
                    t = s
                else:
                    t = jnp.concatenate([s[:, 1:], zrow], axis=1)
                taps9.append(t)
        taps9.append(zeros((Bb, H, W, _L - 9 * C)))
        cat = jnp.concatenate(taps9, axis=3)
        w0c = w0_ref[...]
        strips = []
        for h0 in range(0, H, Hs):
            t2 = cat[:, h0:h0 + Hs].reshape(M, _L)
            y = jnp.dot(t2, w0c, preferred_element_type=f32) + bp_ref[0]
            y = jnp.maximum(y, 0.0)
            strips.append(y.astype(bf16).reshape(Bb, Hs, W, _L))
        return strips[0] if len(strips) == 1 else jnp.concatenate(strips, axis=1)

    bt4 = x_ref.shape[0]
    bt2 = 2 * bt4

    xv = x_ref[...]
    a = conv0(xv)
    a, = conv3x3([a], [wp(0)], [bp_ref[1]], relu=True,
                 fuse_pool=True)

    oa, ob = conv3x3([a], [lambda t: w2c_ref[t]],
                     [bp_ref[2], bp_ref[2]], relu=True)
    a2 = jnp.stack([oa, ob], axis=1).reshape(bt2, 8, 8, _L)
    a2, = conv3x3([a2], [wp(1)], [bp_ref[3]], relu=True,
                  fuse_pool=True)

    flat = a2.reshape(bt2, 16 * _L)
    z = jnp.dot(flat, wenc_ref[...], preferred_element_type=f32) + benc_ref[...]
    o = jnp.dot(z.astype(bf16), wdec_ref[...], preferred_element_type=f32)
    o = jnp.maximum(o + bdec_ref[...], 0.0).astype(bf16)
    a2 = o.reshape(bt2, 4, 4, _L)

    r = a2.reshape(bt4, 2, 4, 4, _L)
    yev, yod = r[:, 0], r[:, 1]
    a = conv_up([yev, yod],
                [lambda p, k: wu4_ref[0, p, k], lambda p, k: wu4_ref[1, p, k]],
                bp_ref[4], relu=True)
    a, = conv3x3([a], [wp(2)], [bp_ref[5]], relu=True)
    a = conv_up([a], [lambda p, k: wu6_ref[p, k]],
                bp_ref[6], relu=True)
    a, = conv3x3([a], [wp(3)], [bp_ref[7]], relu=True)
    a, = conv3x3([a], [wp(4)], [bp_ref[8]], relu=False,
                 out_dtype=f32)

    out_ref[...] = jnp.concatenate(
        [a[..., 0:4], a[..., 64:68], a[..., 128:132], a[..., 192:196]], axis=-1)


def _fold_up(taps):
    dmap = {(0, -1): (0,), (0, 0): (1, 2), (1, 0): (0, 1), (1, 1): (2,)}
    phases = []
    for py in (0, 1):
        for px in (0, 1):
            ktaps = []
            for sy in ((-1, 0) if py == 0 else (0, 1)):
                for sx in ((-1, 0) if px == 0 else (0, 1)):
                    w = 0
                    for dy in dmap[(py, sy)]:
                        for dx in dmap[(px, sx)]:
                            w = w + taps[dy * 3 + dx]
                    ktaps.append(w)
            phases.append(jnp.stack(ktaps))
    return jnp.stack(phases).astype(jnp.bfloat16)


def _pack_params(wconv, bconv, wenc, benc, wdec, bdec):
    bf16 = jnp.bfloat16
    f32 = jnp.float32
    z9 = jnp.zeros((9, _L, _L), bf16)

    def diag4(i):
        w = wconv[i][:, :64, :64]
        out = z9
        for e in range(4):
            out = out.at[:, 64 * e:64 * e + 64, 64 * e:64 * e + 64].set(w)
        return out

    def diag2(i):
        w = wconv[i][:, :128, :128]
        return z9.at[:, 0:128, 0:128].set(w).at[:, 128:256, 128:256].set(w)

    w2 = wconv[2][:, :64, :128]
    w2a = z9.at[:, 0:64, 0:128].set(w2).at[:, 64:128, 128:256].set(w2)
    w2b = z9.at[:, 128:192, 0:128].set(w2).at[:, 192:256, 128:256].set(w2)
    w2c = jnp.concatenate([w2a, w2b], axis=2)
    wp = jnp.stack([diag4(1), diag2(3), diag4(5), diag4(7), diag4(8)])

    z9f = jnp.zeros((9, _L, _L), f32)
    w4 = wconv[4][:, :128, :64].astype(f32)
    w4a = z9f.at[:, 0:128, 0:64].set(w4).at[:, 128:256, 64:128].set(w4)
    w4b = z9f.at[:, 0:128, 128:192].set(w4).at[:, 128:256, 192:256].set(w4)
    wu4 = jnp.stack([_fold_up(w4a), _fold_up(w4b)])
    w6 = wconv[6][:, :64, :64].astype(f32)
    w6d = jnp.zeros((9, _L, _L), f32)
    for e in range(4):
        w6d = w6d.at[:, 64 * e:64 * e + 64, 64 * e:64 * e + 64].set(w6)
    wu6 = _fold_up(w6d)

    w0s = wconv[0][:, :4, :64]
    w0 = jnp.zeros((9, 16, _L), bf16)
    for e in range(4):
        w0 = w0.at[:, 4 * e:4 * e + 4, 64 * e:64 * e + 64].set(w0s)
    w0 = jnp.pad(w0.reshape(9 * 16, _L), ((0, _L - 9 * 16), (0, 0)))

    def tile4(v):
        return jnp.concatenate([v[:64]] * 4)

    def tile2(v):
        return jnp.concatenate([v[:128]] * 2)

    bp = jnp.stack([tile4(bconv[0]), tile4(bconv[1]), tile2(bconv[2]),
                    tile2(bconv[3]), tile4(bconv[4]), tile4(bconv[5]),
                    tile4(bconv[6]), tile4(bconv[7]), tile4(bconv[8])]).astype(f32)

    z16 = jnp.zeros((16, _L, _L), bf16)
    wenc2 = z16.at[:, :128, :128].set(wenc).at[:, 128:, 128:].set(wenc)
    wenc2 = wenc2.reshape(16 * _L, _L)
    benc2 = jnp.concatenate([benc, benc], axis=1).astype(f32)
    wdec2 = z16.at[:, :128, :128].set(wdec).at[:, 128:, 128:].set(wdec)
    wdec2 = wdec2.transpose(1, 0, 2).reshape(_L, 16 * _L)
    bdec2 = jnp.concatenate([bdec, bdec], axis=1).reshape(1, 16 * _L).astype(f32)
    return w0, wp, w2c, wu4, wu6, bp, wenc2, benc2, wdec2, bdec2


def kernel(wconv, bconv, wenc, benc, wdec, bdec, x_nchw):
    B, C, H, W = x_nchw.shape
    for bt in (32, 16, 8, 4):
        if B % bt == 0:
            Bt = bt
            break
    else:
        raise ValueError("batch must be divisible by 4")
    bt4 = Bt // 4

    w0, wp, w2c, wu4, wu6, bp, wenc2, benc2, wdec2, bdec2 = _pack_params(
        wconv, bconv, wenc, benc, wdec, bdec)

    xt = jnp.transpose(x_nchw, (0, 2, 3, 1)).astype(jnp.bfloat16)
    xt = jnp.pad(xt, ((0, 0), (0, 0), (0, 0), (0, 4 - C)))
    x4 = xt.reshape(B // 4, 4, H, W, 4).transpose(0, 2, 3, 1, 4)
    x4 = x4.reshape(B // 4, H, W, 16)

    out = pl.pallas_call(
        _autoenc_body,
        out_shape=jax.ShapeDtypeStruct((B // 4, H, W, 16), jnp.float32),
        grid=(B // Bt,),
        in_specs=[
            pl.BlockSpec((bt4, H, W, 16), lambda b: (b, 0, 0, 0)),
            pl.BlockSpec((_L, _L), lambda b: (0, 0)),
            pl.BlockSpec((5, 9, _L, _L), lambda b: (0, 0, 0, 0)),
            pl.BlockSpec((9, _L, 2 * _L), lambda b: (0, 0, 0)),
            pl.BlockSpec((2, 4, 4, _L, _L), lambda b: (0, 0, 0, 0, 0)),
            pl.BlockSpec((4, 4, _L, _L), lambda b: (0, 0, 0, 0)),
            pl.BlockSpec((9, _L), lambda b: (0, 0)),
            pl.BlockSpec((16 * _L, _L), lambda b: (0, 0)),
            pl.BlockSpec((1, _L), lambda b: (0, 0)),
            pl.BlockSpec((_L, 16 * _L), lambda b: (0, 0)),
            pl.BlockSpec((1, 16 * _L), lambda b: (0, 0)),
        ],
        out_specs=pl.BlockSpec((bt4, H, W, 16), lambda b: (b, 0, 0, 0)),
        compiler_params=pltpu.CompilerParams(
            dimension_semantics=("parallel",),
            vmem_limit_bytes=48 * 1024 * 1024,
        ),
    )(x4, w0, wp, w2c, wu4, wu6, bp, wenc2, benc2, wdec2, bdec2)

    y = out.reshape(B // 4, H, W, 4, 4)[..., :C]
    return y.transpose(0, 3, 4, 1, 2).reshape(B, C, H, W)


# speedup vs baseline: 1.0890x; 1.0012x over previous
import jax
import jax.numpy as jnp
from jax.experimental import pallas as pl
from jax.experimental.pallas import tpu as pltpu

_L = 256
_MAX_M = 256


def _autoenc_body(x_ref, w0_ref, wp_ref, w2c_ref, wu6_ref, bp_ref,
                  wenc_ref, benc_ref, wdec_ref, bdec_ref, out_ref):
    f32 = jnp.float32
    bf16 = jnp.bfloat16

    zcache = {}

    def zeros(shape):
        if shape not in zcache:
            zcache[shape] = jnp.zeros(shape, bf16)
        return zcache[shape]

    def conv3x3(srcs, wgetters, bias, relu, out_dtype=bf16, fuse_pool=False):
        Bb, H, W, C = srcs[0].shape
        Hs = H
        while Hs > 1 and Hs % 2 == 0 and Bb * Hs * W > _MAX_M:
            Hs //= 2
        M = Bb * Hs * W
        zcol = zeros((Bb, H, 1, C))

        shifted = []
        for s in srcs:
            sm = jnp.concatenate([zcol, s[:, :, :W - 1, :]], axis=2)
            sp = jnp.concatenate([s[:, :, 1:, :], zcol], axis=2)
            shifted.append((sm, s, sp))

        def rows(ab, lo):
            hi = lo + Hs
            parts = []
            if lo < 0:
                parts.append(zeros((Bb, -lo, W, C)))
            lo_c, hi_c = max(lo, 0), min(hi, H)
            if hi_c > lo_c:
                parts.append(ab[:, lo_c:hi_c])
            if hi > H:
                parts.append(zeros((Bb, hi - H, W, C)))
            return parts[0] if len(parts) == 1 else jnp.concatenate(parts, axis=1)

        nout = len(bias)
        outs = [[] for _ in bias]
        for h0 in range(0, H, Hs):
            acc = None
            for dy in range(3):
                for si in range(len(srcs)):
                    for dx in range(3):
                        tap = rows(shifted[si][dx], h0 + dy - 1)
                        t2 = tap.reshape(M, C)
                        part = jnp.dot(t2, wgetters[si](dy * 3 + dx),
                                       preferred_element_type=f32)
                        acc = part if acc is None else acc + part
            for j in range(nout):
                y = acc[:, j * _L:(j + 1) * _L] + bias[j]
                if relu:
                    y = jnp.maximum(y, 0.0)
                if fuse_pool:
                    y5 = y.reshape(Bb, Hs // 2, 2, W, _L)
                    v = jnp.maximum(y5[:, :, 0], y5[:, :, 1])
                    s = v.reshape(Bb, Hs // 2, W // 2, 2, _L)
                    y4 = jnp.maximum(s[:, :, :, 0, :], s[:, :, :, 1, :])
                    outs[j].append(y4.astype(out_dtype))
                else:
                    outs[j].append(y.astype(out_dtype).reshape(Bb, Hs, W, _L))
        return [o[0] if len(o) == 1 else jnp.concatenate(o, axis=1) for o in outs]

    def conv_up(srcs, wgetters, bias, relu, phase_out=False):
        Bb, H, W, C = srcs[0].shape
        Hs = H
        while Hs > 1 and Hs % 2 == 0 and Bb * Hs * W > _MAX_M:
            Hs //= 2
        M = Bb * Hs * W
        zcol = zeros((Bb, H, 1, C))
        shifted = []
        for s in srcs:
            sm = jnp.concatenate([zcol, s[:, :, :W - 1, :]], axis=2)
            sp = jnp.concatenate([s[:, :, 1:, :], zcol], axis=2)
            shifted.append((sm, s, sp))

        def rows(ab, lo):
            hi = lo + Hs
            parts = []
            if lo < 0:
                parts.append(zeros((Bb, -lo, W, C)))
            lo_c, hi_c = max(lo, 0), min(hi, H)
            if hi_c > lo_c:
                parts.append(ab[:, lo_c:hi_c])
            if hi > H:
                parts.append(zeros((Bb, hi - H, W, C)))
            return parts[0] if len(parts) == 1 else jnp.concatenate(parts, axis=1)

        phases = []
        for py in (0, 1):
            for px in (0, 1):
                strips = []
                for h0 in range(0, H, Hs):
                    acc = None
                    for syi, sy in enumerate((-1, 0) if py == 0 else (0, 1)):
                        for si in range(len(srcs)):
                            for sxi, sx in enumerate((-1, 0) if px == 0 else (0, 1)):
                                tap = rows(shifted[si][sx + 1], h0 + sy)
                                t2 = tap.reshape(M, C)
                                part = jnp.dot(
                                    t2, wgetters[si](py * 2 + px, syi * 2 + sxi),
                                    preferred_element_type=f32)
                                acc = part if acc is None else acc + part
                    y = acc + bias
                    if relu:
                        y = jnp.maximum(y, 0.0)
                    strips.append(y.astype(bf16).reshape(Bb, Hs, W, _L))
                phases.append(strips[0] if len(strips) == 1
                              else jnp.concatenate(strips, axis=1))
        if phase_out:
            return phases
        p00, p01, p10, p11 = phases

        def ilw(u, v):
            return jnp.stack([u, v], axis=3).reshape(Bb, H, 2 * W, _L)

        r0, r1 = ilw(p00, p01), ilw(p10, p11)
        return jnp.stack([r0, r1], axis=2).reshape(Bb, 2 * H, 2 * W, _L)

    def upsample2x(a):
        Bb, H, W, C = a.shape
        r = jnp.broadcast_to(a[:, :, None], (Bb, H, 2, W, C))
        r = r.reshape(Bb, 2 * H, W, C)
        c = jnp.broadcast_to(r[:, :, :, None, :], (Bb, 2 * H, W, 2, C))
        return c.reshape(Bb, 2 * H, 2 * W, C)

    def wp(i):
        return lambda t, i=i: wp_ref[i, t]

    def conv0(xv):
        Bb, H, W, C = xv.shape
        Hs = H
        while Hs > 1 and Hs % 2 == 0 and Bb * Hs * W > _MAX_M:
            Hs //= 2
        M = Bb * Hs * W
        zcol = zeros((Bb, H, 1, C))
        zrow = zeros((Bb, 1, W, C))
        sm = jnp.concatenate([zcol, xv[:, :, :W - 1, :]], axis=2)
        sp = jnp.concatenate([xv[:, :, 1:, :], zcol], axis=2)
        taps9 = []
        for dy in range(3):
            for s in (sm, xv, sp):
                if dy == 0:
                    t = jnp.concatenate([zrow, s[:, :H - 1]], axis=1)
                elif dy == 1:
                    t = s
                else:
                    t = jnp.concatenate([s[:, 1:], zrow], axis=1)
                taps9.append(t)
        taps9.append(zeros((Bb, H, W, _L - 9 * C)))
        cat = jnp.concatenate(taps9, axis=3)
        w0c = w0_ref[...]
        strips = []
        for h0 in range(0, H, Hs):
            t2 = cat[:, h0:h0 + Hs].reshape(M, _L)
            y = jnp.dot(t2, w0c, preferred_element_type=f32) + bp_ref[0]
            y = jnp.maximum(y, 0.0)
            strips.append(y.astype(bf16).reshape(Bb, Hs, W, _L))
        return strips[0] if len(strips) == 1 else jnp.concatenate(strips, axis=1)

    bt4 = x_ref.shape[0]
    bt2 = 2 * bt4

    xv = x_ref[...]
    a = conv0(xv)
    a, = conv3x3([a], [wp(0)], [bp_ref[1]], relu=True,
                 fuse_pool=True)

    oa, ob = conv3x3([a], [lambda t: w2c_ref[t]],
                     [bp_ref[2], bp_ref[2]], relu=True)
    a2 = jnp.stack([oa, ob], axis=1).reshape(bt2, 8, 8, _L)
    a2, = conv3x3([a2], [wp(1)], [bp_ref[3]], relu=True,
                  fuse_pool=True)

    flat = a2.reshape(bt2, 16 * _L)
    z = jnp.dot(flat, wenc_ref[...], preferred_element_type=f32) + benc_ref[...]
    o = jnp.dot(z.astype(bf16), wdec_ref[...], preferred_element_type=f32)
    o = jnp.maximum(o + bdec_ref[...], 0.0).astype(bf16)
    a2 = o.reshape(bt2, 4, 4, _L)

    a2 = upsample2x(a2)
    r = a2.reshape(bt4, 2, 8, 8, _L)
    yev, yod = r[:, 0], r[:, 1]
    a, = conv3x3([yev, yod], [wp(2), wp(3)], [bp_ref[4]],
                 relu=True)
    a, = conv3x3([a], [wp(4)], [bp_ref[5]], relu=True)
    a = conv_up([a], [lambda p, k: wu6_ref[p, k]],
                bp_ref[6], relu=True)
    a, = conv3x3([a], [wp(5)], [bp_ref[7]], relu=True)
    a, = conv3x3([a], [wp(6)], [bp_ref[8]], relu=False,
                 out_dtype=f32)

    out_ref[...] = jnp.concatenate(
        [a[..., 0:4], a[..., 64:68], a[..., 128:132], a[..., 192:196]], axis=-1)


def _fold_up(taps):
    dmap = {(0, -1): (0,), (0, 0): (1, 2), (1, 0): (0, 1), (1, 1): (2,)}
    phases = []
    for py in (0, 1):
        for px in (0, 1):
            ktaps = []
            for sy in ((-1, 0) if py == 0 else (0, 1)):
                for sx in ((-1, 0) if px == 0 else (0, 1)):
                    w = 0
                    for dy in dmap[(py, sy)]:
                        for dx in dmap[(px, sx)]:
                            w = w + taps[dy * 3 + dx]
                    ktaps.append(w)
            phases.append(jnp.stack(ktaps))
    return jnp.stack(phases).astype(jnp.bfloat16)


def _pack_params(wconv, bconv, wenc, benc, wdec, bdec):
    bf16 = jnp.bfloat16
    f32 = jnp.float32
    z9 = jnp.zeros((9, _L, _L), bf16)

    def diag4(i):
        w = wconv[i][:, :64, :64]
        out = z9
        for e in range(4):
            out = out.at[:, 64 * e:64 * e + 64, 64 * e:64 * e + 64].set(w)
        return out

    def diag2(i):
        w = wconv[i][:, :128, :128]
        return z9.at[:, 0:128, 0:128].set(w).at[:, 128:256, 128:256].set(w)

    w2 = wconv[2][:, :64, :128]
    w2a = z9.at[:, 0:64, 0:128].set(w2).at[:, 64:128, 128:256].set(w2)
    w2b = z9.at[:, 128:192, 0:128].set(w2).at[:, 192:256, 128:256].set(w2)
    w2c = jnp.concatenate([w2a, w2b], axis=2)
    w4 = wconv[4][:, :128, :64]
    w4a = z9.at[:, 0:128, 0:64].set(w4).at[:, 128:256, 64:128].set(w4)
    w4b = z9.at[:, 0:128, 128:192].set(w4).at[:, 128:256, 192:256].set(w4)
    wp = jnp.stack([diag4(1), diag2(3), w4a, w4b,
                    diag4(5), diag4(7), diag4(8)])

    w6 = wconv[6][:, :64, :64].astype(f32)
    w6d = jnp.zeros((9, _L, _L), f32)
    for e in range(4):
        w6d = w6d.at[:, 64 * e:64 * e + 64, 64 * e:64 * e + 64].set(w6)
    wu6 = _fold_up(w6d)

    w0s = wconv[0][:, :4, :64]
    w0 = jnp.zeros((9, 16, _L), bf16)
    for e in range(4):
        w0 = w0.at[:, 4 * e:4 * e + 4, 64 * e:64 * e + 64].set(w0s)
    w0 = jnp.pad(w0.reshape(9 * 16, _L), ((0, _L - 9 * 16), (0, 0)))

    def tile4(v):
        return jnp.concatenate([v[:64]] * 4)

    def tile2(v):
        return jnp.concatenate([v[:128]] * 2)

    bp = jnp.stack([tile4(bconv[0]), tile4(bconv[1]), tile2(bconv[2]),
                    tile2(bconv[3]), tile4(bconv[4]), tile4(bconv[5]),
                    tile4(bconv[6]), tile4(bconv[7]), tile4(bconv[8])]).astype(f32)

    z16 = jnp.zeros((16, _L, _L), bf16)
    wenc2 = z16.at[:, :128, :128].set(wenc).at[:, 128:, 128:].set(wenc)
    wenc2 = wenc2.reshape(16 * _L, _L)
    benc2 = jnp.concatenate([benc, benc], axis=1).astype(f32)
    wdec2 = z16.at[:, :128, :128].set(wdec).at[:, 128:, 128:].set(wdec)
    wdec2 = wdec2.transpose(1, 0, 2).reshape(_L, 16 * _L)
    bdec2 = jnp.concatenate([bdec, bdec], axis=1).reshape(1, 16 * _L).astype(f32)
    return w0, wp, w2c, wu6, bp, wenc2, benc2, wdec2, bdec2


def kernel(wconv, bconv, wenc, benc, wdec, bdec, x_nchw):
    B, C, H, W = x_nchw.shape
    for bt in (32, 16, 8, 4):
        if B % bt == 0:
            Bt = bt
            break
    else:
        raise ValueError("batch must be divisible by 4")
    bt4 = Bt // 4

    w0, wp, w2c, wu6, bp, wenc2, benc2, wdec2, bdec2 = _pack_params(
        wconv, bconv, wenc, benc, wdec, bdec)

    xt = jnp.transpose(x_nchw, (0, 2, 3, 1)).astype(jnp.bfloat16)
    xt = jnp.pad(xt, ((0, 0), (0, 0), (0, 0), (0, 4 - C)))
    x4 = xt.reshape(B // 4, 4, H, W, 4).transpose(0, 2, 3, 1, 4)
    x4 = x4.reshape(B // 4, H, W, 16)

    out = pl.pallas_call(
        _autoenc_body,
        out_shape=jax.ShapeDtypeStruct((B // 4, H, W, 16), jnp.float32),
        grid=(B // Bt,),
        in_specs=[
            pl.BlockSpec((bt4, H, W, 16), lambda b: (b, 0, 0, 0)),
            pl.BlockSpec((_L, _L), lambda b: (0, 0)),
            pl.BlockSpec((7, 9, _L, _L), lambda b: (0, 0, 0, 0)),
            pl.BlockSpec((9, _L, 2 * _L), lambda b: (0, 0, 0)),
            pl.BlockSpec((4, 4, _L, _L), lambda b: (0, 0, 0, 0)),
            pl.BlockSpec((9, _L), lambda b: (0, 0)),
            pl.BlockSpec((16 * _L, _L), lambda b: (0, 0)),
            pl.BlockSpec((1, _L), lambda b: (0, 0)),
            pl.BlockSpec((_L, 16 * _L), lambda b: (0, 0)),
            pl.BlockSpec((1, 16 * _L), lambda b: (0, 0)),
        ],
        out_specs=pl.BlockSpec((bt4, H, W, 16), lambda b: (b, 0, 0, 0)),
        compiler_params=pltpu.CompilerParams(
            dimension_semantics=("parallel",),
            vmem_limit_bytes=48 * 1024 * 1024,
        ),
    )(x4, w0, wp, w2c, wu6, bp, wenc2, benc2, wdec2, bdec2)

    y = out.reshape(B // 4, H, W, 4, 4)[..., :C]
    return y.transpose(0, 3, 4, 1, 2).reshape(B, C, H, W)


# speedup vs baseline: 1.0949x; 1.0055x over previous
import jax
import jax.numpy as jnp
from jax.experimental import pallas as pl
from jax.experimental.pallas import tpu as pltpu

_L = 256
_MAX_M = 256


def _autoenc_body(x_ref, w0_ref, wp_ref, w2c_ref, wu6_ref, bp_ref,
                  wenc_ref, benc_ref, wdec_ref, bdec_ref, out_ref):
    f32 = jnp.float32
    bf16 = jnp.bfloat16

    zcache = {}

    def zeros(shape):
        if shape not in zcache:
            zcache[shape] = jnp.zeros(shape, bf16)
        return zcache[shape]

    def conv3x3(srcs, wgetters, bias, relu, out_dtype=bf16, fuse_pool=False):
        Bb, H, W, C = srcs[0].shape
        Hs = H
        while Hs > 1 and Hs % 2 == 0 and Bb * Hs * W > _MAX_M:
            Hs //= 2
        M = Bb * Hs * W
        zcol = zeros((Bb, H, 1, C))

        shifted = []
        for s in srcs:
            sm = jnp.concatenate([zcol, s[:, :, :W - 1, :]], axis=2)
            sp = jnp.concatenate([s[:, :, 1:, :], zcol], axis=2)
            shifted.append((sm, s, sp))

        def rows(ab, lo):
            hi = lo + Hs
            parts = []
            if lo < 0:
                parts.append(zeros((Bb, -lo, W, C)))
            lo_c, hi_c = max(lo, 0), min(hi, H)
            if hi_c > lo_c:
                parts.append(ab[:, lo_c:hi_c])
            if hi > H:
                parts.append(zeros((Bb, hi - H, W, C)))
            return parts[0] if len(parts) == 1 else jnp.concatenate(parts, axis=1)

        nout = len(bias)
        outs = [[] for _ in bias]
        for h0 in range(0, H, Hs):
            acc = None
            for dy in range(3):
                for si in range(len(srcs)):
                    for dx in range(3):
                        tap = rows(shifted[si][dx], h0 + dy - 1)
                        t2 = tap.reshape(M, C)
                        part = jnp.dot(t2, wgetters[si](dy * 3 + dx),
                                       preferred_element_type=f32)
                        acc = part if acc is None else acc + part
            for j in range(nout):
                y = acc[:, j * _L:(j + 1) * _L] + bias[j]
                if relu:
                    y = jnp.maximum(y, 0.0)
                if fuse_pool:
                    y5 = y.reshape(Bb, Hs // 2, 2, W, _L)
                    v = jnp.maximum(y5[:, :, 0], y5[:, :, 1])
                    s = v.reshape(Bb, Hs // 2, W // 2, 2, _L)
                    y4 = jnp.maximum(s[:, :, :, 0, :], s[:, :, :, 1, :])
                    outs[j].append(y4.astype(out_dtype))
                else:
                    outs[j].append(y.astype(out_dtype).reshape(Bb, Hs, W, _L))
        return [o[0] if len(o) == 1 else jnp.concatenate(o, axis=1) for o in outs]

    def conv_up(srcs, wgetters, bias, relu, phase_out=False):
        Bb, H, W, C = srcs[0].shape
        Hs = H
        while Hs > 1 and Hs % 2 == 0 and Bb * Hs * W > _MAX_M:
            Hs //= 2
        M = Bb * Hs * W
        zcol = zeros((Bb, H, 1, C))
        shifted = []
        for s in srcs:
            sm = jnp.concatenate([zcol, s[:, :, :W - 1, :]], axis=2)
            sp = jnp.concatenate([s[:, :, 1:, :], zcol], axis=2)
            shifted.append((sm, s, sp))

        def rows(ab, lo):
            hi = lo + Hs
            parts = []
            if lo < 0:
                parts.append(zeros((Bb, -lo, W, C)))
            lo_c, hi_c = max(lo, 0), min(hi, H)
            if hi_c > lo_c:
                parts.append(ab[:, lo_c:hi_c])
            if hi > H:
                parts.append(zeros((Bb, hi - H, W, C)))
            return parts[0] if len(parts) == 1 else jnp.concatenate(parts, axis=1)

        phases = []
        for py in (0, 1):
            for px in (0, 1):
                strips = []
                for h0 in range(0, H, Hs):
                    acc = None
                    for syi, sy in enumerate((-1, 0) if py == 0 else (0, 1)):
                        for si in range(len(srcs)):
                            for sxi, sx in enumerate((-1, 0) if px == 0 else (0, 1)):
                                tap = rows(shifted[si][sx + 1], h0 + sy)
                                t2 = tap.reshape(M, C)
                                part = jnp.dot(
                                    t2, wgetters[si](py * 2 + px, syi * 2 + sxi),
                                    preferred_element_type=f32)
                                acc = part if acc is None else acc + part
                    y = acc + bias
                    if relu:
                        y = jnp.maximum(y, 0.0)
                    strips.append(y.astype(bf16).reshape(Bb, Hs, W, _L))
                phases.append(strips[0] if len(strips) == 1
                              else jnp.concatenate(strips, axis=1))
        if phase_out:
            return phases
        p00, p01, p10, p11 = phases

        def ilw(u, v):
            return jnp.stack([u, v], axis=3).reshape(Bb, H, 2 * W, _L)

        r0, r1 = ilw(p00, p01), ilw(p10, p11)
        return jnp.stack([r0, r1], axis=2).reshape(Bb, 2 * H, 2 * W, _L)

    def upsample2x(a):
        Bb, H, W, C = a.shape
        r = jnp.broadcast_to(a[:, :, None], (Bb, H, 2, W, C))
        r = r.reshape(Bb, 2 * H, W, C)
        c = jnp.broadcast_to(r[:, :, :, None, :], (Bb, 2 * H, W, 2, C))
        return c.reshape(Bb, 2 * H, 2 * W, C)

    def wp(i):
        return lambda t, i=i: wp_ref[i, t]

    def conv0(xv):
        Bb, H, W, C = xv.shape
        Hs = H
        while Hs > 1 and Hs % 2 == 0 and Bb * Hs * W > _MAX_M:
            Hs //= 2
        M = Bb * Hs * W
        zcol = zeros((Bb, H, 1, C))
        zrow = zeros((Bb, 1, W, C))
        sm = jnp.concatenate([zcol, xv[:, :, :W - 1, :]], axis=2)
        sp = jnp.concatenate([xv[:, :, 1:, :], zcol], axis=2)
        taps9 = []
        for dy in range(3):
            for s in (sm, xv, sp):
                if dy == 0:
                    t = jnp.concatenate([zrow, s[:, :H - 1]], axis=1)
                elif dy == 1:
                    t = s
                else:
                    t = jnp.concatenate([s[:, 1:], zrow], axis=1)
                taps9.append(t)
        taps9.append(zeros((Bb, H, W, _L - 9 * C)))
        cat = jnp.concatenate(taps9, axis=3)
        w0c = w0_ref[...]
        strips = []
        for h0 in range(0, H, Hs):
            t2 = cat[:, h0:h0 + Hs].reshape(M, _L)
            y = jnp.dot(t2, w0c, preferred_element_type=f32) + bp_ref[0]
            y = jnp.maximum(y, 0.0)
            strips.append(y.astype(bf16).reshape(Bb, Hs, W, _L))
        return strips[0] if len(strips) == 1 else jnp.concatenate(strips, axis=1)

    bt4 = x_ref.shape[0]
    bt2 = 2 * bt4

    xv = x_ref[...]
    a = conv0(xv)
    a, = conv3x3([a], [wp(0)], [bp_ref[1]], relu=True,
                 fuse_pool=True)

    oa, ob = conv3x3([a], [lambda t: w2c_ref[t]],
                     [bp_ref[2], bp_ref[2]], relu=True)
    a2 = jnp.stack([oa, ob], axis=1).reshape(bt2, 8, 8, _L)
    a2, = conv3x3([a2], [wp(1)], [bp_ref[3]], relu=True,
                  fuse_pool=True)

    flat = a2.reshape(bt2, 16 * _L)
    z = jnp.dot(flat, wenc_ref[...], preferred_element_type=f32) + benc_ref[...]
    o = jnp.dot(z.astype(bf16), wdec_ref[...], preferred_element_type=f32)
    o = jnp.maximum(o + bdec_ref[...], 0.0).astype(bf16)
    a2 = o.reshape(bt2, 4, 4, _L)

    a2 = upsample2x(a2)
    r = a2.reshape(bt4, 2, 8, 8, _L)
    yev, yod = r[:, 0], r[:, 1]
    a, = conv3x3([yev, yod], [wp(2), wp(3)], [bp_ref[4]],
                 relu=True)
    a, = conv3x3([a], [wp(4)], [bp_ref[5]], relu=True)
    a = conv_up([a], [lambda p, k: wu6_ref[p, k]],
                bp_ref[6], relu=True)
    a, = conv3x3([a], [wp(5)], [bp_ref[7]], relu=True)
    a, = conv3x3([a], [wp(6)], [bp_ref[8]], relu=False,
                 out_dtype=f32)

    out_ref[...] = jnp.concatenate(
        [a[..., 0:4], a[..., 64:68], a[..., 128:132], a[..., 192:196]], axis=-1)


def _fold_up(taps):
    dmap = {(0, -1): (0,), (0, 0): (1, 2), (1, 0): (0, 1), (1, 1): (2,)}
    phases = []
    for py in (0, 1):
        for px in (0, 1):
            ktaps = []
            for sy in ((-1, 0) if py == 0 else (0, 1)):
                for sx in ((-1, 0) if px == 0 else (0, 1)):
                    w = 0
                    for dy in dmap[(py, sy)]:
                        for dx in dmap[(px, sx)]:
                            w = w + taps[dy * 3 + dx]
                    ktaps.append(w)
            phases.append(jnp.stack(ktaps))
    return jnp.stack(phases).astype(jnp.bfloat16)


def _pack_params(wconv, bconv, wenc, benc, wdec, bdec):
    bf16 = jnp.bfloat16
    f32 = jnp.float32
    z9 = jnp.zeros((9, _L, _L), bf16)

    def diag4(i):
        w = wconv[i][:, :64, :64]
        out = z9
        for e in range(4):
            out = out.at[:, 64 * e:64 * e + 64, 64 * e:64 * e + 64].set(w)
        return out

    def diag2(i):
        w = wconv[i][:, :128, :128]
        return z9.at[:, 0:128, 0:128].set(w).at[:, 128:256, 128:256].set(w)

    w2 = wconv[2][:, :64, :128]
    w2a = z9.at[:, 0:64, 0:128].set(w2).at[:, 64:128, 128:256].set(w2)
    w2b = z9.at[:, 128:192, 0:128].set(w2).at[:, 192:256, 128:256].set(w2)
    w2c = jnp.concatenate([w2a, w2b], axis=2)
    w4 = wconv[4][:, :128, :64]
    w4a = z9.at[:, 0:128, 0:64].set(w4).at[:, 128:256, 64:128].set(w4)
    w4b = z9.at[:, 0:128, 128:192].set(w4).at[:, 128:256, 192:256].set(w4)
    wp = jnp.stack([diag4(1), diag2(3), w4a, w4b,
                    diag4(5), diag4(7), diag4(8)])

    w6 = wconv[6][:, :64, :64].astype(f32)
    w6d = jnp.zeros((9, _L, _L), f32)
    for e in range(4):
        w6d = w6d.at[:, 64 * e:64 * e + 64, 64 * e:64 * e + 64].set(w6)
    wu6 = _fold_up(w6d)

    w0s = wconv[0][:, :4, :64]
    w0 = jnp.zeros((9, 16, _L), bf16)
    for e in range(4):
        w0 = w0.at[:, 4 * e:4 * e + 4, 64 * e:64 * e + 64].set(w0s)
    w0 = jnp.pad(w0.reshape(9 * 16, _L), ((0, _L - 9 * 16), (0, 0)))

    def tile4(v):
        return jnp.concatenate([v[:64]] * 4)

    def tile2(v):
        return jnp.concatenate([v[:128]] * 2)

    bp = jnp.stack([tile4(bconv[0]), tile4(bconv[1]), tile2(bconv[2]),
                    tile2(bconv[3]), tile4(bconv[4]), tile4(bconv[5]),
                    tile4(bconv[6]), tile4(bconv[7]), tile4(bconv[8])]).astype(f32)

    z16 = jnp.zeros((16, _L, _L), bf16)
    wenc2 = z16.at[:, :128, :128].set(wenc).at[:, 128:, 128:].set(wenc)
    wenc2 = wenc2.reshape(16 * _L, _L)
    benc2 = jnp.concatenate([benc, benc], axis=1).astype(f32)
    wdec2 = z16.at[:, :128, :128].set(wdec).at[:, 128:, 128:].set(wdec)
    wdec2 = wdec2.transpose(1, 0, 2).reshape(_L, 16 * _L)
    bdec2 = jnp.concatenate([bdec, bdec], axis=1).reshape(1, 16 * _L).astype(f32)
    return w0, wp, w2c, wu6, bp, wenc2, benc2, wdec2, bdec2


def kernel(wconv, bconv, wenc, benc, wdec, bdec, x_nchw):
    B, C, H, W = x_nchw.shape
    for bt in (32, 16, 8, 4):
        if B % bt == 0:
            Bt = bt
            break
    else:
        raise ValueError("batch must be divisible by 4")
    bt4 = Bt // 4

    w0, wp, w2c, wu6, bp, wenc2, benc2, wdec2, bdec2 = _pack_params(
        wconv, bconv, wenc, benc, wdec, bdec)

    xt = jnp.transpose(x_nchw, (0, 2, 3, 1)).astype(jnp.bfloat16)
    xt = jnp.pad(xt, ((0, 0), (0, 0), (0, 0), (0, 4 - C)))
    x4 = xt.reshape(B // 4, 4, H, W, 4).transpose(0, 2, 3, 1, 4)
    x4 = x4.reshape(B // 4, H, W, 16)

    out = pl.pallas_call(
        _autoenc_body,
        out_shape=jax.ShapeDtypeStruct((B // 4, H, W, 16), jnp.float32),
        grid=(B // Bt,),
        in_specs=[
            pl.BlockSpec((bt4, H, W, 16), lambda b: (b, 0, 0, 0)),
            pl.BlockSpec((_L, _L), lambda b: (0, 0)),
            pl.BlockSpec((7, 9, _L, _L), lambda b: (0, 0, 0, 0)),
            pl.BlockSpec((9, _L, 2 * _L), lambda b: (0, 0, 0)),
            pl.BlockSpec((4, 4, _L, _L), lambda b: (0, 0, 0, 0)),
            pl.BlockSpec((9, _L), lambda b: (0, 0)),
            pl.BlockSpec((16 * _L, _L), lambda b: (0, 0)),
            pl.BlockSpec((1, _L), lambda b: (0, 0)),
            pl.BlockSpec((_L, 16 * _L), lambda b: (0, 0)),
            pl.BlockSpec((1, 16 * _L), lambda b: (0, 0)),
        ],
        out_specs=pl.BlockSpec((bt4, H, W, 16), lambda b: (b, 0, 0, 0)),
        compiler_params=pltpu.CompilerParams(
            dimension_semantics=("parallel",),
            vmem_limit_bytes=64 * 1024 * 1024,
        ),
    )(x4, w0, wp, w2c, wu6, bp, wenc2, benc2, wdec2, bdec2)

    y = out.reshape(B // 4, H, W, 4, 4)[..., :C]
    return y.transpose(0, 3, 4, 1, 2).reshape(B, C, H, W)
